# baseline (speedup 1.0000x reference)
import functools

import jax
import jax.numpy as jnp
from jax import lax
from jax.experimental import pallas as pl
from jax.experimental.pallas import tpu as pltpu
from jax.experimental.pallas import tpu_sc as plsc

_QMAX = 127.0

_N_TOK = 2048
_TOP_K = 2
_N_EXP = 8
_D_MODEL = 1024
_N_S = _N_TOK * _TOP_K
_BLK = 128
_NB = _N_S // _BLK
_N_WORK = _NB + _N_EXP

_CHUNK = 64
_TCHUNK = 32


def _excl_cumsum8(x):
    s = jnp.concatenate([jnp.zeros((1, 1), x.dtype), x[:, :-1]], axis=1)
    for sh in (1, 2, 4):
        s = s + jnp.concatenate(
            [jnp.zeros((1, sh), x.dtype), s[:, :-sh]], axis=1)
    return s


def _routing_kernel(idx_ref, pos_ref, wl_ref):
    idx = idx_ref[...]
    e_iota = lax.broadcasted_iota(jnp.int32, (_N_TOK, _N_EXP), 1)
    oh0 = (idx[:, 0:1] == e_iota).astype(jnp.float32)
    oh1 = (idx[:, 1:2] == e_iota).astype(jnp.float32)
    both = oh0 + oh1

    chunk = 512
    tri = (lax.broadcasted_iota(jnp.int32, (chunk, chunk), 0)
           > lax.broadcasted_iota(jnp.int32, (chunk, chunk), 1)
           ).astype(jnp.bfloat16)
    carry = jnp.zeros((1, _N_EXP), jnp.float32)
    cparts = []
    for c in range(_N_TOK // chunk):
        blk = both[c * chunk:(c + 1) * chunk]
        cparts.append(
            lax.dot_general(tri, blk.astype(jnp.bfloat16),
                            (((1,), (0,)), ((), ())),
                            preferred_element_type=jnp.float32) + carry)
        carry = carry + jnp.sum(blk, axis=0, keepdims=True)
    cum = jnp.concatenate(cparts, axis=0)
    counts = carry

    off = _excl_cumsum8(counts)

    base = cum + off
    pos0 = jnp.sum(oh0 * base, axis=1, keepdims=True)
    pos1 = (jnp.sum(oh1 * base, axis=1, keepdims=True)
            + (idx[:, 0:1] == idx[:, 1:2]).astype(jnp.float32))
    pos_ref[...] = jnp.concatenate([pos0, pos1], axis=1).astype(jnp.int32)

    counts_i = counts.astype(jnp.int32)
    off_i = off.astype(jnp.int32)
    ends_i = off_i + counts_i
    bstart = off_i // _BLK
    bend = (ends_i + (_BLK - 1)) // _BLK
    nb = jnp.where(counts_i > 0, bend - bstart, 0)
    ws = _excl_cumsum8(nb)
    total = jnp.sum(nb)

    w_iota = lax.broadcasted_iota(jnp.int32, (_N_WORK, _N_EXP), 0)
    e_row = lax.broadcasted_iota(jnp.int32, (_N_WORK, _N_EXP), 1)
    sel = ((w_iota >= ws) & (w_iota < ws + nb)).astype(jnp.int32)
    expert_of = jnp.sum(sel * e_row, axis=1, keepdims=True)
    block_of = jnp.sum(sel * (bstart + (w_iota - ws)), axis=1, keepdims=True)
    rstart = jnp.sum(sel * off_i, axis=1, keepdims=True)
    rend = jnp.sum(sel * ends_i, axis=1, keepdims=True)

    emax = jnp.max(jnp.where(counts_i > 0, e_iota[0:1], 0),
                   axis=1, keepdims=True)
    valid = w_iota[:, 0:1] < total
    block_of = jnp.where(valid, block_of, _NB - 1)
    expert_of = jnp.where(valid, expert_of, emax)
    rstart = jnp.where(valid, rstart, 0)
    rend = jnp.where(valid, rend, 0)
    prev = jnp.concatenate(
        [jnp.full((1, 1), -1, jnp.int32), block_of[:-1]], axis=0)
    first = ((block_of != prev) & valid).astype(jnp.int32)
    eprev = jnp.concatenate(
        [jnp.full((1, 1), -1, jnp.int32), expert_of[:-1]], axis=0)
    newe = (expert_of != eprev).astype(jnp.int32)

    zero = jnp.zeros((_N_WORK, 1), jnp.int32)
    wl_ref[...] = jnp.concatenate(
        [block_of, expert_of, rstart, rend, first, newe, zero, zero], axis=1)


def _routing_call(top_k_index):
    return pl.pallas_call(
        _routing_kernel,
        out_shape=[
            jax.ShapeDtypeStruct((_N_TOK, _TOP_K), jnp.int32),
            jax.ShapeDtypeStruct((_N_WORK, _N_EXP), jnp.int32),
        ],
    )(top_k_index)


def _make_dispatch():
    info = plsc.get_sparse_core_info()
    nw = info.num_cores * info.num_subcores
    per_w = _N_S // nw
    mesh = plsc.VectorSubcoreMesh(core_axis_name="c", subcore_axis_name="s")

    @functools.partial(
        pl.kernel, mesh=mesh,
        out_type=jax.ShapeDtypeStruct((_N_S, _D_MODEL), jnp.float32),
        scratch_types=[
            pltpu.VMEM((_CHUNK,), jnp.int32),
            pltpu.VMEM((_CHUNK,), jnp.int32),
            pltpu.VMEM((_CHUNK, _D_MODEL), jnp.float32),
            pltpu.SemaphoreType.DMA,
        ],
    )
    def dispatch(hid_hbm, pos_hbm, tok_hbm, xs_hbm, tok_v, pos_v, rows_v, sem):
        wid = lax.axis_index("s") * info.num_cores + lax.axis_index("c")
        for c in range(per_w // _CHUNK):
            base = wid * per_w + c * _CHUNK
            pltpu.sync_copy(tok_hbm.at[pl.ds(base, _CHUNK)], tok_v)
            pltpu.sync_copy(pos_hbm.at[pl.ds(base, _CHUNK)], pos_v)
            pltpu.async_copy(hid_hbm.at[tok_v], rows_v, sem).wait()
            pltpu.async_copy(rows_v, xs_hbm.at[pos_v], sem).wait()

    return dispatch


def _fq_bf16(x):
    s = jnp.max(jnp.abs(x), axis=-1, keepdims=True) / _QMAX
    s = jnp.where(s <= 0.0, 1.0, s)
    return (jnp.round(x / s) * s).astype(jnp.bfloat16)


def _gemm_kernel(wl_ref, x_ref, gu_ref, dn_ref, y_ref, gu_s, dn_s):
    w = pl.program_id(0)

    @pl.when(wl_ref[w, 5] == 1)
    def _():
        gu_s[...] = gu_ref[0].astype(jnp.bfloat16)
        dn_s[...] = dn_ref[0].astype(jnp.bfloat16)

    q1 = _fq_bf16(x_ref[...])
    h = lax.dot_general(q1, gu_s[...], (((1,), (0,)), ((), ())),
                        preferred_element_type=jnp.float32)
    f = h.shape[-1] // 2
    gate = h[:, :f]
    up = h[:, f:]
    g = gate * jax.nn.sigmoid(gate) * up
    q2 = _fq_bf16(g)
    y = lax.dot_general(q2, dn_s[...], (((1,), (0,)), ((), ())),
                        preferred_element_type=jnp.float32)

    rows = wl_ref[w, 0] * _BLK + lax.broadcasted_iota(jnp.int32, (_BLK, 1), 0)
    m = ((rows >= wl_ref[w, 2]) & (rows < wl_ref[w, 3])).astype(jnp.float32)

    @pl.when(wl_ref[w, 4] == 1)
    def _():
        y_ref[...] = m * y

    @pl.when(wl_ref[w, 4] == 0)
    def _():
        y_ref[...] += m * y


def _gemm_call(wl, x_sorted, gu, dn):
    grid_spec = pltpu.PrefetchScalarGridSpec(
        num_scalar_prefetch=1,
        grid=(_N_WORK,),
        in_specs=[
            pl.BlockSpec((_BLK, _D_MODEL), lambda w, wl: (wl[w, 0], 0)),
            pl.BlockSpec((1, _D_MODEL, 2 * _D_MODEL),
                         lambda w, wl: (wl[w, 1], 0, 0)),
            pl.BlockSpec((1, _D_MODEL, _D_MODEL),
                         lambda w, wl: (wl[w, 1], 0, 0)),
        ],
        out_specs=pl.BlockSpec((_BLK, _D_MODEL), lambda w, wl: (wl[w, 0], 0)),
        scratch_shapes=[
            pltpu.VMEM((_D_MODEL, 2 * _D_MODEL), jnp.bfloat16),
            pltpu.VMEM((_D_MODEL, _D_MODEL), jnp.bfloat16),
        ],
    )
    return pl.pallas_call(
        _gemm_kernel,
        grid_spec=grid_spec,
        out_shape=jax.ShapeDtypeStruct((_N_S, _D_MODEL), jnp.float32),
        compiler_params=pltpu.CompilerParams(
            dimension_semantics=("arbitrary",),
        ),
    )(wl, x_sorted, gu, dn)


def _make_combine():
    info = plsc.get_sparse_core_info()
    nw = info.num_cores * info.num_subcores
    tok_per_w = _N_TOK // nw
    schunk = 2 * _TCHUNK
    lanes = info.num_lanes
    mesh = plsc.VectorSubcoreMesh(core_axis_name="c", subcore_axis_name="s")

    @functools.partial(
        pl.kernel, mesh=mesh,
        out_type=jax.ShapeDtypeStruct((_N_TOK, _D_MODEL), jnp.float32),
        scratch_types=[
            pltpu.VMEM((schunk,), jnp.int32),
            pltpu.VMEM((schunk, lanes), jnp.float32),
            pltpu.VMEM((schunk, _D_MODEL), jnp.float32),
            pltpu.VMEM((_TCHUNK, _D_MODEL), jnp.float32),
            pltpu.SemaphoreType.DMA,
        ],
    )
    def combine(y_hbm, pos_hbm, wt_hbm, out_hbm, pos_v, w_v, rows_v, out_v,
                sem):
        wid = lax.axis_index("s") * info.num_cores + lax.axis_index("c")

        for c in range(tok_per_w // _TCHUNK):
            sbase = wid * 2 * tok_per_w + c * schunk
            tbase = wid * tok_per_w + c * _TCHUNK
            pltpu.sync_copy(pos_hbm.at[pl.ds(sbase, schunk)], pos_v)
            pltpu.sync_copy(wt_hbm.at[pl.ds(sbase, schunk)], w_v)
            pltpu.async_copy(y_hbm.at[pos_v], rows_v, sem).wait()

            def tbody(t, _):
                w0 = w_v[2 * t, :]
                w1 = w_v[2 * t + 1, :]

                def jbody(j, _):
                    sl = pl.ds(j * lanes, lanes)
                    out_v[t, sl] = (w0 * rows_v[2 * t, sl]
                                    + w1 * rows_v[2 * t + 1, sl])
                    return 0

                lax.fori_loop(0, _D_MODEL // lanes, jbody, 0)
                return 0

            lax.fori_loop(0, _TCHUNK, tbody, 0)
            pltpu.sync_copy(out_v, out_hbm.at[pl.ds(tbase, _TCHUNK)])

    return combine


def kernel(hidden_states, top_k_index, top_k_weights, gate_up_proj, down_proj):
    top_k_index = top_k_index.astype(jnp.int32)

    pos, wl = _routing_call(top_k_index)
    pos_flat = pos.reshape(_N_S)
    tok_rep = jnp.repeat(jnp.arange(_N_TOK, dtype=jnp.int32), _TOP_K)
    w_bcast = jnp.broadcast_to(top_k_weights.reshape(_N_S, 1), (_N_S, 16))

    x_sorted = _make_dispatch()(hidden_states, pos_flat, tok_rep)
    y = _gemm_call(wl, x_sorted, gate_up_proj, down_proj)
    out = _make_combine()(y, pos_flat, w_bcast)
    return out


# speedup vs baseline: 1.0424x; 1.0424x over previous
import functools

import jax
import jax.numpy as jnp
from jax import lax
from jax.experimental import pallas as pl
from jax.experimental.pallas import tpu as pltpu
from jax.experimental.pallas import tpu_sc as plsc

_QMAX = 127.0

_N_TOK = 2048
_TOP_K = 2
_N_EXP = 8
_D_MODEL = 1024
_N_S = _N_TOK * _TOP_K
_BLK = 128
_NB = _N_S // _BLK
_N_WORK = _NB + _N_EXP

_CHUNK = 64
_TCHUNK = 16


def _excl_cumsum8(x):
    s = jnp.concatenate([jnp.zeros((1, 1), x.dtype), x[:, :-1]], axis=1)
    for sh in (1, 2, 4):
        s = s + jnp.concatenate(
            [jnp.zeros((1, sh), x.dtype), s[:, :-sh]], axis=1)
    return s


def _routing_kernel(idx_ref, pos_ref, wl_ref):
    idx = idx_ref[...]
    e_iota = lax.broadcasted_iota(jnp.int32, (_N_TOK, _N_EXP), 1)
    oh0 = (idx[:, 0:1] == e_iota).astype(jnp.float32)
    oh1 = (idx[:, 1:2] == e_iota).astype(jnp.float32)
    both = oh0 + oh1

    chunk = 512
    tri = (lax.broadcasted_iota(jnp.int32, (chunk, chunk), 0)
           > lax.broadcasted_iota(jnp.int32, (chunk, chunk), 1)
           ).astype(jnp.bfloat16)
    carry = jnp.zeros((1, _N_EXP), jnp.float32)
    cparts = []
    for c in range(_N_TOK // chunk):
        blk = both[c * chunk:(c + 1) * chunk]
        cparts.append(
            lax.dot_general(tri, blk.astype(jnp.bfloat16),
                            (((1,), (0,)), ((), ())),
                            preferred_element_type=jnp.float32) + carry)
        carry = carry + jnp.sum(blk, axis=0, keepdims=True)
    cum = jnp.concatenate(cparts, axis=0)
    counts = carry

    off = _excl_cumsum8(counts)

    base = cum + off
    pos0 = jnp.sum(oh0 * base, axis=1, keepdims=True)
    pos1 = (jnp.sum(oh1 * base, axis=1, keepdims=True)
            + (idx[:, 0:1] == idx[:, 1:2]).astype(jnp.float32))
    pos_ref[...] = jnp.concatenate([pos0, pos1], axis=1).astype(jnp.int32)

    counts_i = counts.astype(jnp.int32)
    off_i = off.astype(jnp.int32)
    ends_i = off_i + counts_i
    bstart = off_i // _BLK
    bend = (ends_i + (_BLK - 1)) // _BLK
    nb = jnp.where(counts_i > 0, bend - bstart, 0)
    ws = _excl_cumsum8(nb)
    total = jnp.sum(nb)

    w_iota = lax.broadcasted_iota(jnp.int32, (_N_WORK, _N_EXP), 0)
    e_row = lax.broadcasted_iota(jnp.int32, (_N_WORK, _N_EXP), 1)
    sel = ((w_iota >= ws) & (w_iota < ws + nb)).astype(jnp.int32)
    expert_of = jnp.sum(sel * e_row, axis=1, keepdims=True)
    block_of = jnp.sum(sel * (bstart + (w_iota - ws)), axis=1, keepdims=True)
    rstart = jnp.sum(sel * off_i, axis=1, keepdims=True)
    rend = jnp.sum(sel * ends_i, axis=1, keepdims=True)

    emax = jnp.max(jnp.where(counts_i > 0, e_iota[0:1], 0),
                   axis=1, keepdims=True)
    valid = w_iota[:, 0:1] < total
    block_of = jnp.where(valid, block_of, _NB - 1)
    expert_of = jnp.where(valid, expert_of, emax)
    rstart = jnp.where(valid, rstart, 0)
    rend = jnp.where(valid, rend, 0)
    prev = jnp.concatenate(
        [jnp.full((1, 1), -1, jnp.int32), block_of[:-1]], axis=0)
    first = ((block_of != prev) & valid).astype(jnp.int32)
    eprev = jnp.concatenate(
        [jnp.full((1, 1), -1, jnp.int32), expert_of[:-1]], axis=0)
    newe = (expert_of != eprev).astype(jnp.int32)

    zero = jnp.zeros((_N_WORK, 1), jnp.int32)
    wl_ref[...] = jnp.concatenate(
        [block_of, expert_of, rstart, rend, first, newe, zero, zero], axis=1)


def _routing_call(top_k_index):
    return pl.pallas_call(
        _routing_kernel,
        out_shape=[
            jax.ShapeDtypeStruct((_N_TOK, _TOP_K), jnp.int32),
            jax.ShapeDtypeStruct((_N_WORK, _N_EXP), jnp.int32),
        ],
    )(top_k_index)


def _make_dispatch():
    info = plsc.get_sparse_core_info()
    nw = info.num_cores * info.num_subcores
    tok_per_w = _N_TOK // nw
    mesh = plsc.VectorSubcoreMesh(core_axis_name="c", subcore_axis_name="s")

    @functools.partial(
        pl.kernel, mesh=mesh,
        out_type=jax.ShapeDtypeStruct((_N_S, _D_MODEL), jnp.float32),
        scratch_types=[
            pltpu.VMEM((tok_per_w,), jnp.int32),
            pltpu.VMEM((tok_per_w,), jnp.int32),
            pltpu.VMEM((tok_per_w, _D_MODEL), jnp.float32),
            pltpu.SemaphoreType.DMA,
            pltpu.SemaphoreType.DMA,
        ],
    )
    def dispatch(hid_hbm, pos0_hbm, pos1_hbm, xs_hbm, p0_v, p1_v, rows_v,
                 gsem, ssem):
        wid = lax.axis_index("s") * info.num_cores + lax.axis_index("c")
        base = wid * tok_per_w
        pltpu.sync_copy(pos0_hbm.at[pl.ds(base, tok_per_w)], p0_v)
        pltpu.sync_copy(pos1_hbm.at[pl.ds(base, tok_per_w)], p1_v)
        pltpu.async_copy(hid_hbm.at[pl.ds(base, tok_per_w)], rows_v,
                         gsem).wait()
        s0 = pltpu.async_copy(rows_v, xs_hbm.at[p0_v], ssem)
        s1 = pltpu.async_copy(rows_v, xs_hbm.at[p1_v], ssem)
        s0.wait()
        s1.wait()

    return dispatch


def _fq_bf16(x):
    s = jnp.max(jnp.abs(x), axis=-1, keepdims=True) / _QMAX
    s = jnp.where(s <= 0.0, 1.0, s)
    return (jnp.round(x / s) * s).astype(jnp.bfloat16)


def _gemm_kernel(wl_ref, x_ref, gu_ref, dn_ref, y_ref, gu_s, dn_s):
    w = pl.program_id(0)

    @pl.when(wl_ref[w, 5] == 1)
    def _():
        gu_s[...] = gu_ref[0].astype(jnp.bfloat16)
        dn_s[...] = dn_ref[0].astype(jnp.bfloat16)

    q1 = _fq_bf16(x_ref[...])
    h = lax.dot_general(q1, gu_s[...], (((1,), (0,)), ((), ())),
                        preferred_element_type=jnp.float32)
    f = h.shape[-1] // 2
    gate = h[:, :f]
    up = h[:, f:]
    g = gate * jax.nn.sigmoid(gate) * up
    q2 = _fq_bf16(g)
    y = lax.dot_general(q2, dn_s[...], (((1,), (0,)), ((), ())),
                        preferred_element_type=jnp.float32)

    rows = wl_ref[w, 0] * _BLK + lax.broadcasted_iota(jnp.int32, (_BLK, 1), 0)
    m = ((rows >= wl_ref[w, 2]) & (rows < wl_ref[w, 3])).astype(jnp.float32)

    @pl.when(wl_ref[w, 4] == 1)
    def _():
        y_ref[...] = m * y

    @pl.when(wl_ref[w, 4] == 0)
    def _():
        y_ref[...] += m * y


def _gemm_call(wl, x_sorted, gu, dn):
    grid_spec = pltpu.PrefetchScalarGridSpec(
        num_scalar_prefetch=1,
        grid=(_N_WORK,),
        in_specs=[
            pl.BlockSpec((_BLK, _D_MODEL), lambda w, wl: (wl[w, 0], 0)),
            pl.BlockSpec((1, _D_MODEL, 2 * _D_MODEL),
                         lambda w, wl: (wl[w, 1], 0, 0)),
            pl.BlockSpec((1, _D_MODEL, _D_MODEL),
                         lambda w, wl: (wl[w, 1], 0, 0)),
        ],
        out_specs=pl.BlockSpec((_BLK, _D_MODEL), lambda w, wl: (wl[w, 0], 0)),
        scratch_shapes=[
            pltpu.VMEM((_D_MODEL, 2 * _D_MODEL), jnp.bfloat16),
            pltpu.VMEM((_D_MODEL, _D_MODEL), jnp.bfloat16),
        ],
    )
    return pl.pallas_call(
        _gemm_kernel,
        grid_spec=grid_spec,
        out_shape=jax.ShapeDtypeStruct((_N_S, _D_MODEL), jnp.float32),
        compiler_params=pltpu.CompilerParams(
            dimension_semantics=("arbitrary",),
        ),
    )(wl, x_sorted, gu, dn)


def _make_combine():
    info = plsc.get_sparse_core_info()
    nw = info.num_cores * info.num_subcores
    tok_per_w = _N_TOK // nw
    schunk = 2 * _TCHUNK
    lanes = info.num_lanes
    mesh = plsc.VectorSubcoreMesh(core_axis_name="c", subcore_axis_name="s")

    nchunk = tok_per_w // _TCHUNK

    @functools.partial(
        pl.kernel, mesh=mesh,
        out_type=jax.ShapeDtypeStruct((_N_TOK, _D_MODEL), jnp.float32),
        scratch_types=[
            pltpu.VMEM((nchunk, schunk), jnp.int32),
            pltpu.VMEM((nchunk, schunk, lanes), jnp.float32),
            pltpu.VMEM((2, schunk, _D_MODEL), jnp.float32),
            pltpu.VMEM((_TCHUNK, _D_MODEL), jnp.float32),
            pltpu.SemaphoreType.DMA,
            pltpu.SemaphoreType.DMA,
        ],
    )
    def combine(y_hbm, pos_hbm, wt_hbm, out_hbm, pos_v, w_v, rows_v, out_v,
                sem0, sem1):
        wid = lax.axis_index("s") * info.num_cores + lax.axis_index("c")
        sems = [sem0, sem1]

        for c in range(nchunk):
            sbase = wid * 2 * tok_per_w + c * schunk
            pltpu.sync_copy(pos_hbm.at[pl.ds(sbase, schunk)], pos_v.at[c])
            pltpu.sync_copy(wt_hbm.at[pl.ds(sbase, schunk)], w_v.at[c])

        gathers = [None] * nchunk
        gathers[0] = pltpu.async_copy(
            y_hbm.at[pos_v.at[0]], rows_v.at[0], sems[0])
        for c in range(nchunk):
            if c + 1 < nchunk:
                gathers[c + 1] = pltpu.async_copy(
                    y_hbm.at[pos_v.at[c + 1]], rows_v.at[(c + 1) % 2],
                    sems[(c + 1) % 2])
            gathers[c].wait()
            tbase = wid * tok_per_w + c * _TCHUNK
            rbuf = rows_v.at[c % 2]

            def tbody(t, _):
                w0 = w_v[c, 2 * t, :]
                w1 = w_v[c, 2 * t + 1, :]

                def jbody(j, _):
                    sl = pl.ds(j * lanes, lanes)
                    out_v[t, sl] = (w0 * rbuf[2 * t, sl]
                                    + w1 * rbuf[2 * t + 1, sl])
                    return 0

                lax.fori_loop(0, _D_MODEL // lanes, jbody, 0)
                return 0

            lax.fori_loop(0, _TCHUNK, tbody, 0)
            pltpu.sync_copy(out_v, out_hbm.at[pl.ds(tbase, _TCHUNK)])

    return combine


def kernel(hidden_states, top_k_index, top_k_weights, gate_up_proj, down_proj):
    top_k_index = top_k_index.astype(jnp.int32)

    pos, wl = _routing_call(top_k_index)
    pos_flat = pos.reshape(_N_S)
    w_bcast = jnp.broadcast_to(top_k_weights.reshape(_N_S, 1), (_N_S, 16))

    x_sorted = _make_dispatch()(hidden_states, pos[:, 0], pos[:, 1])
    y = _gemm_call(wl, x_sorted, gate_up_proj, down_proj)
    out = _make_combine()(y, pos_flat, w_bcast)
    return out


# speedup vs baseline: 1.0921x; 1.0477x over previous
import functools

import jax
import jax.numpy as jnp
from jax import lax
from jax.experimental import pallas as pl
from jax.experimental.pallas import tpu as pltpu
from jax.experimental.pallas import tpu_sc as plsc

_QMAX = 127.0

_N_TOK = 2048
_TOP_K = 2
_N_EXP = 8
_D_MODEL = 1024
_N_S = _N_TOK * _TOP_K
_BLK = 256
_NB = _N_S // _BLK
_N_WORK = _NB + _N_EXP

_CHUNK = 64
_TCHUNK = 16


def _excl_cumsum8(x):
    s = jnp.concatenate([jnp.zeros((1, 1), x.dtype), x[:, :-1]], axis=1)
    for sh in (1, 2, 4):
        s = s + jnp.concatenate(
            [jnp.zeros((1, sh), x.dtype), s[:, :-sh]], axis=1)
    return s


def _routing_kernel(idx_ref, pos_ref, wl_ref):
    idx = idx_ref[...]
    e_iota = lax.broadcasted_iota(jnp.int32, (_N_TOK, _N_EXP), 1)
    oh0 = (idx[:, 0:1] == e_iota).astype(jnp.float32)
    oh1 = (idx[:, 1:2] == e_iota).astype(jnp.float32)
    both = oh0 + oh1

    chunk = 512
    tri = (lax.broadcasted_iota(jnp.int32, (chunk, chunk), 0)
           > lax.broadcasted_iota(jnp.int32, (chunk, chunk), 1)
           ).astype(jnp.bfloat16)
    carry = jnp.zeros((1, _N_EXP), jnp.float32)
    cparts = []
    for c in range(_N_TOK // chunk):
        blk = both[c * chunk:(c + 1) * chunk]
        cparts.append(
            lax.dot_general(tri, blk.astype(jnp.bfloat16),
                            (((1,), (0,)), ((), ())),
                            preferred_element_type=jnp.float32) + carry)
        carry = carry + jnp.sum(blk, axis=0, keepdims=True)
    cum = jnp.concatenate(cparts, axis=0)
    counts = carry

    off = _excl_cumsum8(counts)

    base = cum + off
    pos0 = jnp.sum(oh0 * base, axis=1, keepdims=True)
    pos1 = (jnp.sum(oh1 * base, axis=1, keepdims=True)
            + (idx[:, 0:1] == idx[:, 1:2]).astype(jnp.float32))
    pos_ref[...] = jnp.concatenate([pos0, pos1], axis=1).astype(jnp.int32)

    counts_i = counts.astype(jnp.int32)
    off_i = off.astype(jnp.int32)
    ends_i = off_i + counts_i
    bstart = off_i // _BLK
    bend = (ends_i + (_BLK - 1)) // _BLK
    nb = jnp.where(counts_i > 0, bend - bstart, 0)
    ws = _excl_cumsum8(nb)
    total = jnp.sum(nb)

    w_iota = lax.broadcasted_iota(jnp.int32, (_N_WORK, _N_EXP), 0)
    e_row = lax.broadcasted_iota(jnp.int32, (_N_WORK, _N_EXP), 1)
    sel = ((w_iota >= ws) & (w_iota < ws + nb)).astype(jnp.int32)
    expert_of = jnp.sum(sel * e_row, axis=1, keepdims=True)
    block_of = jnp.sum(sel * (bstart + (w_iota - ws)), axis=1, keepdims=True)
    rstart = jnp.sum(sel * off_i, axis=1, keepdims=True)
    rend = jnp.sum(sel * ends_i, axis=1, keepdims=True)

    emax = jnp.max(jnp.where(counts_i > 0, e_iota[0:1], 0),
                   axis=1, keepdims=True)
    valid = w_iota[:, 0:1] < total
    block_of = jnp.where(valid, block_of, _NB - 1)
    expert_of = jnp.where(valid, expert_of, emax)
    rstart = jnp.where(valid, rstart, 0)
    rend = jnp.where(valid, rend, 0)
    prev = jnp.concatenate(
        [jnp.full((1, 1), -1, jnp.int32), block_of[:-1]], axis=0)
    first = ((block_of != prev) & valid).astype(jnp.int32)
    eprev = jnp.concatenate(
        [jnp.full((1, 1), -1, jnp.int32), expert_of[:-1]], axis=0)
    newe = (expert_of != eprev).astype(jnp.int32)

    zero = jnp.zeros((_N_WORK, 1), jnp.int32)
    wl_ref[...] = jnp.concatenate(
        [block_of, expert_of, rstart, rend, first, newe, zero, zero], axis=1)


def _routing_call(top_k_index):
    return pl.pallas_call(
        _routing_kernel,
        out_shape=[
            jax.ShapeDtypeStruct((_N_TOK, _TOP_K), jnp.int32),
            jax.ShapeDtypeStruct((_N_WORK, _N_EXP), jnp.int32),
        ],
    )(top_k_index)


def _make_dispatch():
    info = plsc.get_sparse_core_info()
    nw = info.num_cores * info.num_subcores
    tok_per_w = _N_TOK // nw
    mesh = plsc.VectorSubcoreMesh(core_axis_name="c", subcore_axis_name="s")

    @functools.partial(
        pl.kernel, mesh=mesh,
        out_type=jax.ShapeDtypeStruct((_N_S, _D_MODEL), jnp.float32),
        scratch_types=[
            pltpu.VMEM((tok_per_w,), jnp.int32),
            pltpu.VMEM((tok_per_w,), jnp.int32),
            pltpu.VMEM((tok_per_w, _D_MODEL), jnp.float32),
            pltpu.SemaphoreType.DMA,
            pltpu.SemaphoreType.DMA,
        ],
    )
    def dispatch(hid_hbm, pos0_hbm, pos1_hbm, xs_hbm, p0_v, p1_v, rows_v,
                 gsem, ssem):
        wid = lax.axis_index("s") * info.num_cores + lax.axis_index("c")
        base = wid * tok_per_w
        pltpu.sync_copy(pos0_hbm.at[pl.ds(base, tok_per_w)], p0_v)
        pltpu.sync_copy(pos1_hbm.at[pl.ds(base, tok_per_w)], p1_v)
        pltpu.async_copy(hid_hbm.at[pl.ds(base, tok_per_w)], rows_v,
                         gsem).wait()
        s0 = pltpu.async_copy(rows_v, xs_hbm.at[p0_v], ssem)
        s1 = pltpu.async_copy(rows_v, xs_hbm.at[p1_v], ssem)
        s0.wait()
        s1.wait()

    return dispatch


def _fq_bf16(x):
    s = jnp.max(jnp.abs(x), axis=-1, keepdims=True) / _QMAX
    s = jnp.where(s <= 0.0, 1.0, s)
    return (jnp.round(x / s) * s).astype(jnp.bfloat16)


def _gemm_kernel(wl_ref, x_ref, gu_ref, dn_ref, y_ref, gu_s, dn_s):
    w = pl.program_id(0)

    @pl.when(wl_ref[w, 5] == 1)
    def _():
        gu_s[...] = gu_ref[0].astype(jnp.bfloat16)
        dn_s[...] = dn_ref[0].astype(jnp.bfloat16)

    q1 = _fq_bf16(x_ref[...])
    h = lax.dot_general(q1, gu_s[...], (((1,), (0,)), ((), ())),
                        preferred_element_type=jnp.float32)
    f = h.shape[-1] // 2
    gate = h[:, :f]
    up = h[:, f:]
    g = gate * jax.nn.sigmoid(gate) * up
    q2 = _fq_bf16(g)
    y = lax.dot_general(q2, dn_s[...], (((1,), (0,)), ((), ())),
                        preferred_element_type=jnp.float32)

    rows = wl_ref[w, 0] * _BLK + lax.broadcasted_iota(jnp.int32, (_BLK, 1), 0)
    m = ((rows >= wl_ref[w, 2]) & (rows < wl_ref[w, 3])).astype(jnp.float32)

    @pl.when(wl_ref[w, 4] == 1)
    def _():
        y_ref[...] = m * y

    @pl.when(wl_ref[w, 4] == 0)
    def _():
        y_ref[...] += m * y


def _gemm_call(wl, x_sorted, gu, dn):
    grid_spec = pltpu.PrefetchScalarGridSpec(
        num_scalar_prefetch=1,
        grid=(_N_WORK,),
        in_specs=[
            pl.BlockSpec((_BLK, _D_MODEL), lambda w, wl: (wl[w, 0], 0)),
            pl.BlockSpec((1, _D_MODEL, 2 * _D_MODEL),
                         lambda w, wl: (wl[w, 1], 0, 0)),
            pl.BlockSpec((1, _D_MODEL, _D_MODEL),
                         lambda w, wl: (wl[w, 1], 0, 0)),
        ],
        out_specs=pl.BlockSpec((_BLK, _D_MODEL), lambda w, wl: (wl[w, 0], 0)),
        scratch_shapes=[
            pltpu.VMEM((_D_MODEL, 2 * _D_MODEL), jnp.bfloat16),
            pltpu.VMEM((_D_MODEL, _D_MODEL), jnp.bfloat16),
        ],
    )
    return pl.pallas_call(
        _gemm_kernel,
        grid_spec=grid_spec,
        out_shape=jax.ShapeDtypeStruct((_N_S, _D_MODEL), jnp.float32),
        compiler_params=pltpu.CompilerParams(
            dimension_semantics=("arbitrary",),
        ),
    )(wl, x_sorted, gu, dn)


def _make_combine():
    info = plsc.get_sparse_core_info()
    nw = info.num_cores * info.num_subcores
    tok_per_w = _N_TOK // nw
    schunk = 2 * _TCHUNK
    lanes = info.num_lanes
    mesh = plsc.VectorSubcoreMesh(core_axis_name="c", subcore_axis_name="s")

    nchunk = tok_per_w // _TCHUNK

    @functools.partial(
        pl.kernel, mesh=mesh,
        out_type=jax.ShapeDtypeStruct((_N_TOK, _D_MODEL), jnp.float32),
        scratch_types=[
            pltpu.VMEM((nchunk, schunk), jnp.int32),
            pltpu.VMEM((nchunk, schunk, lanes), jnp.float32),
            pltpu.VMEM((2, schunk, _D_MODEL), jnp.float32),
            pltpu.VMEM((_TCHUNK, _D_MODEL), jnp.float32),
            pltpu.SemaphoreType.DMA,
            pltpu.SemaphoreType.DMA,
        ],
    )
    def combine(y_hbm, pos_hbm, wt_hbm, out_hbm, pos_v, w_v, rows_v, out_v,
                sem0, sem1):
        wid = lax.axis_index("s") * info.num_cores + lax.axis_index("c")
        sems = [sem0, sem1]

        for c in range(nchunk):
            sbase = wid * 2 * tok_per_w + c * schunk
            pltpu.sync_copy(pos_hbm.at[pl.ds(sbase, schunk)], pos_v.at[c])
            pltpu.sync_copy(wt_hbm.at[pl.ds(sbase, schunk)], w_v.at[c])

        gathers = [None] * nchunk
        gathers[0] = pltpu.async_copy(
            y_hbm.at[pos_v.at[0]], rows_v.at[0], sems[0])
        for c in range(nchunk):
            if c + 1 < nchunk:
                gathers[c + 1] = pltpu.async_copy(
                    y_hbm.at[pos_v.at[c + 1]], rows_v.at[(c + 1) % 2],
                    sems[(c + 1) % 2])
            gathers[c].wait()
            tbase = wid * tok_per_w + c * _TCHUNK
            rbuf = rows_v.at[c % 2]

            def tbody(t, _):
                w0 = w_v[c, 2 * t, :]
                w1 = w_v[c, 2 * t + 1, :]
                for j in range(_D_MODEL // lanes):
                    sl = pl.ds(j * lanes, lanes)
                    out_v[t, sl] = (w0 * rbuf[2 * t, sl]
                                    + w1 * rbuf[2 * t + 1, sl])
                return 0

            lax.fori_loop(0, _TCHUNK, tbody, 0)
            pltpu.sync_copy(out_v, out_hbm.at[pl.ds(tbase, _TCHUNK)])

    return combine


def kernel(hidden_states, top_k_index, top_k_weights, gate_up_proj, down_proj):
    top_k_index = top_k_index.astype(jnp.int32)

    pos, wl = _routing_call(top_k_index)
    pos_flat = pos.reshape(_N_S)
    w_bcast = jnp.broadcast_to(top_k_weights.reshape(_N_S, 1), (_N_S, 16))

    x_sorted = _make_dispatch()(hidden_states, pos[:, 0], pos[:, 1])
    y = _gemm_call(wl, x_sorted, gate_up_proj, down_proj)
    out = _make_combine()(y, pos_flat, w_bcast)
    return out
